# baseline (speedup 1.0000x reference)
import functools

import jax
from jax import lax
import jax.numpy as jnp
from jax.experimental import pallas as pl
from jax.experimental.pallas import tpu as pltpu
from jax.experimental.pallas import tpu_sc as plsc

_NC, _NS = 2, 16
_NW = _NC * _NS
_CHUNK = 80
_PAD_BLK = 5000


def _pad_body(x_ref, o_ref):
    depth = x_ref.shape[1]
    o_ref[:, :depth] = x_ref[...]
    o_ref[:, depth:] = jnp.zeros_like(x_ref)


def kernel(texts_tokenized, table):
    batch, seq = texts_tokenized.shape
    vocab, depth = table.shape
    num_idx = batch * seq
    b_per_w = num_idx // _NW
    n_chunks = b_per_w // _CHUNK
    assert b_per_w % _CHUNK == 0 and n_chunks % 2 == 0
    assert _CHUNK % 16 == 0

    ids = texts_tokenized.reshape(num_idx)
    tab_pad = pl.pallas_call(
        _pad_body,
        out_shape=jax.ShapeDtypeStruct((vocab, 2 * depth), table.dtype),
        grid=(vocab // _PAD_BLK,),
        in_specs=[pl.BlockSpec((_PAD_BLK, depth), lambda i: (i, 0))],
        out_specs=pl.BlockSpec((_PAD_BLK, 2 * depth), lambda i: (i, 0)),
    )(table)

    mesh = plsc.VectorSubcoreMesh(core_axis_name="c", subcore_axis_name="s")

    @functools.partial(
        pl.kernel,
        mesh=mesh,
        out_type=jax.ShapeDtypeStruct((num_idx, depth), table.dtype),
        scratch_types=[
            pltpu.VMEM((_CHUNK,), jnp.int32),
            pltpu.VMEM((_CHUNK,), jnp.int32),
            pltpu.VMEM((_CHUNK, 2 * depth), table.dtype),
            pltpu.VMEM((_CHUNK, 2 * depth), table.dtype),
            pltpu.VMEM((_CHUNK, depth), table.dtype),
            pltpu.VMEM((_CHUNK, depth), table.dtype),
            pltpu.SemaphoreType.DMA,
            pltpu.SemaphoreType.DMA,
            pltpu.SemaphoreType.DMA,
            pltpu.SemaphoreType.DMA,
            pltpu.SemaphoreType.DMA,
            pltpu.SemaphoreType.DMA,
        ],
    )
    def gather_kernel(tp_hbm, ids_hbm, out_hbm,
                      ids0, ids1, rows0, rows1, out0, out1,
                      semI0, semI1, semG0, semG1, semO0, semO1):
        wid = lax.axis_index("s") * _NC + lax.axis_index("c")
        base = wid * b_per_w

        def start_idx(ci, ids_v, sem):
            pltpu.async_copy(ids_hbm.at[pl.ds(base + ci * _CHUNK, _CHUNK)],
                             ids_v, sem)

        def wait_idx(ids_v, sem):
            pltpu.make_async_copy(ids_hbm.at[pl.ds(base, _CHUNK)],
                                  ids_v, sem).wait()

        def start_gather(ids_v, rows_v, sem):
            pltpu.async_copy(
                tp_hbm.at[plsc.Indices(ids_v, ignored_value=-1)], rows_v, sem)

        def wait_gather(ids_v, rows_v, sem):
            pltpu.make_async_copy(
                tp_hbm.at[plsc.Indices(ids_v, ignored_value=-1)],
                rows_v, sem).wait()

        def compact(rows_v, out_v):
            @functools.partial(plsc.parallel_loop, 0, _CHUNK, unroll=4)
            def _(i):
                for q in range(depth // 16):
                    out_v[i, pl.ds(q * 16, 16)] = rows_v[i, pl.ds(q * 16, 16)]

        def start_out(ci, out_v, sem):
            pltpu.async_copy(out_v, out_hbm.at[pl.ds(base + ci * _CHUNK,
                                                     _CHUNK)], sem)

        def wait_out(out_v, sem):
            pltpu.make_async_copy(out_v, out_hbm.at[pl.ds(base, _CHUNK)],
                                  sem).wait()

        start_idx(0, ids0, semI0)
        wait_idx(ids0, semI0)
        start_gather(ids0, rows0, semG0)
        start_idx(1, ids1, semI1)

        @pl.loop(0, n_chunks // 2)
        def _(gi):
            g = gi * 2

            wait_gather(ids0, rows0, semG0)
            wait_idx(ids1, semI1)
            start_gather(ids1, rows1, semG1)

            @pl.when(g + 2 < n_chunks)
            def _():
                start_idx(g + 2, ids0, semI0)

            @pl.when(g >= 2)
            def _():
                wait_out(out0, semO0)

            compact(rows0, out0)
            start_out(g, out0, semO0)

            wait_gather(ids1, rows1, semG1)

            @pl.when(g + 2 < n_chunks)
            def _():
                wait_idx(ids0, semI0)
                start_gather(ids0, rows0, semG0)

            @pl.when(g + 3 < n_chunks)
            def _():
                start_idx(g + 3, ids1, semI1)

            @pl.when(g >= 2)
            def _():
                wait_out(out1, semO1)

            compact(rows1, out1)
            start_out(g + 1, out1, semO1)

        wait_out(out0, semO0)
        wait_out(out1, semO1)

    out = gather_kernel(tab_pad, ids)
    return out.reshape(batch, seq, depth)


# speedup vs baseline: 1.0898x; 1.0898x over previous
import functools

import jax
from jax import lax
import jax.numpy as jnp
from jax.experimental import pallas as pl
from jax.experimental.pallas import tpu as pltpu
from jax.experimental.pallas import tpu_sc as plsc

_NC, _NS = 2, 16
_NW = _NC * _NS
_CHUNK = 160
_PAD_BLK = 5000


def _pad_body(x_ref, o_ref):
    depth = x_ref.shape[1]
    o_ref[:, :depth] = x_ref[...]
    o_ref[:, depth:] = jnp.zeros_like(x_ref)


def kernel(texts_tokenized, table):
    batch, seq = texts_tokenized.shape
    vocab, depth = table.shape
    num_idx = batch * seq
    b_per_w = num_idx // _NW
    n_chunks = b_per_w // _CHUNK
    assert b_per_w % _CHUNK == 0 and n_chunks % 2 == 0
    assert _CHUNK % 16 == 0

    ids = texts_tokenized.reshape(num_idx)
    tab_pad = pl.pallas_call(
        _pad_body,
        out_shape=jax.ShapeDtypeStruct((vocab, 2 * depth), table.dtype),
        grid=(vocab // _PAD_BLK,),
        in_specs=[pl.BlockSpec((_PAD_BLK, depth), lambda i: (i, 0))],
        out_specs=pl.BlockSpec((_PAD_BLK, 2 * depth), lambda i: (i, 0)),
    )(table)

    mesh = plsc.VectorSubcoreMesh(core_axis_name="c", subcore_axis_name="s")

    @functools.partial(
        pl.kernel,
        mesh=mesh,
        out_type=jax.ShapeDtypeStruct((num_idx, depth), table.dtype),
        scratch_types=[
            pltpu.VMEM((_CHUNK,), jnp.int32),
            pltpu.VMEM((_CHUNK,), jnp.int32),
            pltpu.VMEM((_CHUNK, 2 * depth), table.dtype),
            pltpu.VMEM((_CHUNK, 2 * depth), table.dtype),
            pltpu.VMEM((_CHUNK, depth), table.dtype),
            pltpu.VMEM((_CHUNK, depth), table.dtype),
            pltpu.SemaphoreType.DMA,
            pltpu.SemaphoreType.DMA,
            pltpu.SemaphoreType.DMA,
            pltpu.SemaphoreType.DMA,
            pltpu.SemaphoreType.DMA,
            pltpu.SemaphoreType.DMA,
        ],
    )
    def gather_kernel(tp_hbm, ids_hbm, out_hbm,
                      ids0, ids1, rows0, rows1, out0, out1,
                      semI0, semI1, semG0, semG1, semO0, semO1):
        wid = lax.axis_index("s") * _NC + lax.axis_index("c")
        base = wid * b_per_w

        def start_idx(ci, ids_v, sem):
            pltpu.async_copy(ids_hbm.at[pl.ds(base + ci * _CHUNK, _CHUNK)],
                             ids_v, sem)

        def wait_idx(ids_v, sem):
            pltpu.make_async_copy(ids_hbm.at[pl.ds(base, _CHUNK)],
                                  ids_v, sem).wait()

        def start_gather(ids_v, rows_v, sem):
            pltpu.async_copy(
                tp_hbm.at[plsc.Indices(ids_v, ignored_value=-1)], rows_v, sem)

        def wait_gather(ids_v, rows_v, sem):
            pltpu.make_async_copy(
                tp_hbm.at[plsc.Indices(ids_v, ignored_value=-1)],
                rows_v, sem).wait()

        def compact(rows_v, out_v):
            @pl.loop(0, _CHUNK)
            def _(i):
                for q in range(depth // 16):
                    out_v[i, pl.ds(q * 16, 16)] = rows_v[i, pl.ds(q * 16, 16)]

        def start_out(ci, out_v, sem):
            pltpu.async_copy(out_v, out_hbm.at[pl.ds(base + ci * _CHUNK,
                                                     _CHUNK)], sem)

        def wait_out(out_v, sem):
            pltpu.make_async_copy(out_v, out_hbm.at[pl.ds(base, _CHUNK)],
                                  sem).wait()

        start_idx(0, ids0, semI0)
        wait_idx(ids0, semI0)
        start_gather(ids0, rows0, semG0)
        start_idx(1, ids1, semI1)

        @pl.loop(0, n_chunks // 2)
        def _(gi):
            g = gi * 2

            wait_gather(ids0, rows0, semG0)
            wait_idx(ids1, semI1)
            start_gather(ids1, rows1, semG1)

            @pl.when(g + 2 < n_chunks)
            def _():
                start_idx(g + 2, ids0, semI0)

            @pl.when(g >= 2)
            def _():
                wait_out(out0, semO0)

            compact(rows0, out0)
            start_out(g, out0, semO0)

            wait_gather(ids1, rows1, semG1)

            @pl.when(g + 2 < n_chunks)
            def _():
                wait_idx(ids0, semI0)
                start_gather(ids0, rows0, semG0)

            @pl.when(g + 3 < n_chunks)
            def _():
                start_idx(g + 3, ids1, semI1)

            @pl.when(g >= 2)
            def _():
                wait_out(out1, semO1)

            compact(rows1, out1)
            start_out(g + 1, out1, semO1)

        wait_out(out0, semO0)
        wait_out(out1, semO1)

    out = gather_kernel(tab_pad, ids)
    return out.reshape(batch, seq, depth)
